# baseline (speedup 1.0000x reference)
import jax
import jax.numpy as jnp
from jax import lax
from jax.experimental import pallas as pl
from jax.experimental.pallas import tpu as pltpu

_K = 40
_KP = 64
_BN = 512
_BC = 512
_NEG = -1e30
_HIGH = lax.Precision.HIGHEST


def _xsum_body(x_ref, o_ref):
    i = pl.program_id(1)
    s = jnp.sum(x_ref[0], axis=0, keepdims=True)

    @pl.when(i == 0)
    def _():
        o_ref[0] = s

    @pl.when(i != 0)
    def _():
        o_ref[0] = o_ref[0] + s


def _qsum_body(n_rows, xs_ref, wq_ref, bq_ref, o_ref):
    xs = xs_ref[:, 0, :]
    r = lax.dot_general(xs, wq_ref[...], (((1,), (1,)), ((), ())),
                        precision=_HIGH)
    o_ref[:, 0, :] = r + n_rows * bq_ref[...]


def _t_body(qs_ref, wk_ref, o_ref):
    qs = qs_ref[:, 0, :]
    o_ref[:, 0, :] = lax.dot_general(qs, wk_ref[...], (((1,), (0,)), ((), ())),
                                     precision=_HIGH)


def _score_body(x_ref, t_ref, o_ref):
    o_ref[0] = lax.dot_general(t_ref[0], x_ref[0], (((1,), (1,)), ((), ())),
                               precision=_HIGH)


def _topk_body(n_ctx, s_ref, idx_ref):
    b = pl.program_id(0)
    s = s_ref[0]
    iota = lax.broadcasted_iota(jnp.int32, s.shape, 1)
    for i in range(_K):
        m = jnp.max(s)
        am = jnp.min(jnp.where(s == m, iota, n_ctx))
        idx_ref[b, i] = am
        s = jnp.where(iota == am, _NEG, s)


def _gather_body(idx_ref, x_ref, o_ref):
    del idx_ref
    j = pl.program_id(1)

    @pl.when(j < _K)
    def _():
        o_ref[...] = x_ref[...]

    @pl.when(j >= _K)
    def _():
        o_ref[...] = jnp.zeros_like(o_ref)


def _sel_body(xsel_ref, wk_ref, bk_ref, o_ref):
    r = lax.dot_general(xsel_ref[0], wk_ref[...], (((1,), (1,)), ((), ())),
                        precision=_HIGH)
    o_ref[0] = r + bk_ref[...]


def _p_body(sel_ref, wq_ref, wk_ref, bq_ref, bk_ref,
            pq_ref, pk_ref, aq_ref, ak_ref):
    j = pl.program_id(1)
    s = sel_ref[0]
    pq_ref[0] = lax.dot_general(s, wq_ref[...], (((1,), (0,)), ((), ())),
                                precision=_HIGH)
    pk_ref[0] = lax.dot_general(s, wk_ref[...], (((1,), (0,)), ((), ())),
                                precision=_HIGH)

    @pl.when(j == 0)
    def _():
        kio = lax.broadcasted_iota(jnp.int32, (1, _KP), 1)
        av = lax.dot_general(bq_ref[...], s, (((1,), (1,)), ((), ())),
                             precision=_HIGH)
        aq_ref[0] = jnp.where(kio < _K, av, _NEG)
        akv = lax.dot_general(bk_ref[...], s, (((1,), (1,)), ((), ())),
                              precision=_HIGH)
        ak_ref[0] = jnp.where(kio < _K, akv, _NEG)


def _main_body(c_dim, x_ref, pq_ref, pk_ref, aq_ref, ak_ref,
               a_ref, g_ref, gs_ref):
    i = pl.program_id(1)
    xb = x_ref[0]
    scale = jnp.sqrt(jnp.asarray(c_dim, dtype=jnp.float32))

    s1 = lax.dot_general(xb, pq_ref[0], (((1,), (1,)), ((), ())),
                         precision=_HIGH)
    l1 = (s1 + aq_ref[0]) / scale
    l1 = l1 - jnp.max(l1, axis=1, keepdims=True)
    e1 = jnp.exp(l1)
    a_ref[0] = e1 / jnp.sum(e1, axis=1, keepdims=True)

    s2 = lax.dot_general(xb, pk_ref[0], (((1,), (1,)), ((), ())),
                         precision=_HIGH)
    l2 = (s2 + ak_ref[0]) / scale
    l2 = l2 - jnp.max(l2, axis=1, keepdims=True)
    e2 = jnp.exp(l2)
    bm = e2 / jnp.sum(e2, axis=1, keepdims=True)

    g = lax.dot_general(bm, xb, (((0,), (0,)), ((), ())), precision=_HIGH)
    gs = jnp.sum(bm, axis=0, keepdims=True)

    @pl.when(i == 0)
    def _():
        g_ref[0] = g
        gs_ref[0] = gs

    @pl.when(i != 0)
    def _():
        g_ref[0] = g_ref[0] + g
        gs_ref[0] = gs_ref[0] + gs


def _m_body(g_ref, gs_ref, wv_ref, bv_ref, o_ref):
    m = lax.dot_general(g_ref[0], wv_ref[...], (((1,), (1,)), ((), ())),
                        precision=_HIGH)
    outer = lax.dot_general(gs_ref[0], bv_ref[...], (((0,), (0,)), ((), ())),
                            precision=_HIGH)
    o_ref[0] = m + outer


def _m2_body(m_ref, wo_ref, o_ref):
    o_ref[0] = lax.dot_general(m_ref[0], wo_ref[...], (((1,), (1,)), ((), ())),
                               precision=_HIGH)


def _out_body(x_ref, a_ref, m2_ref, bo_ref, o_ref):
    r = lax.dot_general(a_ref[0], m2_ref[0], (((1,), (0,)), ((), ())),
                        precision=_HIGH)
    o_ref[0] = x_ref[0] + r + bo_ref[...]


def kernel(query, Wq, bq, Wk, bk, Wv, bv, Wo, bo):
    B, N, C = query.shape
    f32 = jnp.float32
    x = query
    bq2 = bq.reshape(1, C)
    bk2 = bk.reshape(1, C)
    bv2 = bv.reshape(1, C)
    bo2 = bo.reshape(1, C)
    nb = N // _BN
    cb = C // _BC

    xsum = pl.pallas_call(
        _xsum_body,
        grid=(B, nb),
        in_specs=[pl.BlockSpec((1, _BN, C), lambda b, i: (b, i, 0))],
        out_specs=pl.BlockSpec((1, 1, C), lambda b, i: (b, 0, 0)),
        out_shape=jax.ShapeDtypeStruct((B, 1, C), f32),
    )(x)

    qsum = pl.pallas_call(
        lambda *a: _qsum_body(float(N), *a),
        grid=(cb,),
        in_specs=[
            pl.BlockSpec((B, 1, C), lambda j: (0, 0, 0)),
            pl.BlockSpec((_BC, C), lambda j: (j, 0)),
            pl.BlockSpec((1, _BC), lambda j: (0, j)),
        ],
        out_specs=pl.BlockSpec((B, 1, _BC), lambda j: (0, 0, j)),
        out_shape=jax.ShapeDtypeStruct((B, 1, C), f32),
    )(xsum, Wq, bq2)

    t = pl.pallas_call(
        _t_body,
        grid=(cb,),
        in_specs=[
            pl.BlockSpec((B, 1, C), lambda j: (0, 0, 0)),
            pl.BlockSpec((C, _BC), lambda j: (0, j)),
        ],
        out_specs=pl.BlockSpec((B, 1, _BC), lambda j: (0, 0, j)),
        out_shape=jax.ShapeDtypeStruct((B, 1, C), f32),
    )(qsum, Wk)

    score = pl.pallas_call(
        _score_body,
        grid=(B, nb),
        in_specs=[
            pl.BlockSpec((1, _BN, C), lambda b, i: (b, i, 0)),
            pl.BlockSpec((1, 1, C), lambda b, i: (b, 0, 0)),
        ],
        out_specs=pl.BlockSpec((1, 1, _BN), lambda b, i: (b, 0, i)),
        out_shape=jax.ShapeDtypeStruct((B, 1, N), f32),
    )(x, t)

    idx = pl.pallas_call(
        lambda *a: _topk_body(N, *a),
        grid=(B,),
        in_specs=[pl.BlockSpec((1, 1, N), lambda b: (b, 0, 0))],
        out_specs=pl.BlockSpec(memory_space=pltpu.SMEM),
        out_shape=jax.ShapeDtypeStruct((B, _K), jnp.int32),
    )(score)

    x3d = x.reshape(B * N, 1, C)
    xsel3d = pl.pallas_call(
        _gather_body,
        grid_spec=pltpu.PrefetchScalarGridSpec(
            num_scalar_prefetch=1,
            grid=(B, _KP),
            in_specs=[
                pl.BlockSpec(
                    (1, 1, C),
                    lambda b, j, idx_ref: (
                        b * N + idx_ref[b, jnp.minimum(j, _K - 1)],
                        0,
                        0,
                    ),
                ),
            ],
            out_specs=pl.BlockSpec(
                (1, 1, C), lambda b, j, idx_ref: (b * _KP + j, 0, 0)),
        ),
        out_shape=jax.ShapeDtypeStruct((B * _KP, 1, C), f32),
    )(idx, x3d)
    xsel = xsel3d.reshape(B, _KP, C)

    sel = pl.pallas_call(
        _sel_body,
        grid=(B, cb),
        in_specs=[
            pl.BlockSpec((1, _KP, C), lambda b, j: (b, 0, 0)),
            pl.BlockSpec((_BC, C), lambda b, j: (j, 0)),
            pl.BlockSpec((1, _BC), lambda b, j: (0, j)),
        ],
        out_specs=pl.BlockSpec((1, _KP, _BC), lambda b, j: (b, 0, j)),
        out_shape=jax.ShapeDtypeStruct((B, _KP, C), f32),
    )(xsel, Wk, bk2)

    pq, pk, aq, ak = pl.pallas_call(
        _p_body,
        grid=(B, cb),
        in_specs=[
            pl.BlockSpec((1, _KP, C), lambda b, j: (b, 0, 0)),
            pl.BlockSpec((C, _BC), lambda b, j: (0, j)),
            pl.BlockSpec((C, _BC), lambda b, j: (0, j)),
            pl.BlockSpec((1, C), lambda b, j: (0, 0)),
            pl.BlockSpec((1, C), lambda b, j: (0, 0)),
        ],
        out_specs=[
            pl.BlockSpec((1, _KP, _BC), lambda b, j: (b, 0, j)),
            pl.BlockSpec((1, _KP, _BC), lambda b, j: (b, 0, j)),
            pl.BlockSpec((1, 1, _KP), lambda b, j: (b, 0, 0)),
            pl.BlockSpec((1, 1, _KP), lambda b, j: (b, 0, 0)),
        ],
        out_shape=[
            jax.ShapeDtypeStruct((B, _KP, C), f32),
            jax.ShapeDtypeStruct((B, _KP, C), f32),
            jax.ShapeDtypeStruct((B, 1, _KP), f32),
            jax.ShapeDtypeStruct((B, 1, _KP), f32),
        ],
    )(sel, Wq, Wk, bq2, bk2)

    a_mat, g_mat, gsum = pl.pallas_call(
        lambda *a: _main_body(C, *a),
        grid=(B, nb),
        in_specs=[
            pl.BlockSpec((1, _BN, C), lambda b, i: (b, i, 0)),
            pl.BlockSpec((1, _KP, C), lambda b, i: (b, 0, 0)),
            pl.BlockSpec((1, _KP, C), lambda b, i: (b, 0, 0)),
            pl.BlockSpec((1, 1, _KP), lambda b, i: (b, 0, 0)),
            pl.BlockSpec((1, 1, _KP), lambda b, i: (b, 0, 0)),
        ],
        out_specs=[
            pl.BlockSpec((1, _BN, _KP), lambda b, i: (b, i, 0)),
            pl.BlockSpec((1, _KP, C), lambda b, i: (b, 0, 0)),
            pl.BlockSpec((1, 1, _KP), lambda b, i: (b, 0, 0)),
        ],
        out_shape=[
            jax.ShapeDtypeStruct((B, N, _KP), f32),
            jax.ShapeDtypeStruct((B, _KP, C), f32),
            jax.ShapeDtypeStruct((B, 1, _KP), f32),
        ],
    )(x, pq, pk, aq, ak)

    m_mat = pl.pallas_call(
        _m_body,
        grid=(B, cb),
        in_specs=[
            pl.BlockSpec((1, _KP, C), lambda b, j: (b, 0, 0)),
            pl.BlockSpec((1, 1, _KP), lambda b, j: (b, 0, 0)),
            pl.BlockSpec((_BC, C), lambda b, j: (j, 0)),
            pl.BlockSpec((1, _BC), lambda b, j: (0, j)),
        ],
        out_specs=pl.BlockSpec((1, _KP, _BC), lambda b, j: (b, 0, j)),
        out_shape=jax.ShapeDtypeStruct((B, _KP, C), f32),
    )(g_mat, gsum, Wv, bv2)

    m2 = pl.pallas_call(
        _m2_body,
        grid=(B, cb),
        in_specs=[
            pl.BlockSpec((1, _KP, C), lambda b, j: (b, 0, 0)),
            pl.BlockSpec((_BC, C), lambda b, j: (j, 0)),
        ],
        out_specs=pl.BlockSpec((1, _KP, _BC), lambda b, j: (b, 0, j)),
        out_shape=jax.ShapeDtypeStruct((B, _KP, C), f32),
    )(m_mat, Wo)

    out = pl.pallas_call(
        _out_body,
        grid=(B, nb),
        in_specs=[
            pl.BlockSpec((1, _BN, C), lambda b, i: (b, i, 0)),
            pl.BlockSpec((1, _BN, _KP), lambda b, i: (b, i, 0)),
            pl.BlockSpec((1, _KP, C), lambda b, i: (b, 0, 0)),
            pl.BlockSpec((1, C), lambda b, i: (0, 0)),
        ],
        out_specs=pl.BlockSpec((1, _BN, C), lambda b, i: (b, i, 0)),
        out_shape=jax.ShapeDtypeStruct((B, N, C), f32),
    )(x, a_mat, m2, bo2)

    return out
